# baseline (speedup 1.0000x reference)
import functools

import jax
import jax.numpy as jnp
from jax import lax
from jax.experimental import pallas as pl
from jax.experimental.pallas import tpu as pltpu
from jax.experimental.pallas import tpu_sc as plsc

_NC = 2
_NS = 16
_PHASE = 512
_RD_TILES = 8


def _make_sc_copy(rows, d, dtype):
    rows_per_core = rows // _NC
    nph = rows_per_core // _PHASE
    rows_rd = _PHASE // _RD_TILES
    rows_wr = _PHASE // (_NS - _RD_TILES)
    mesh = plsc.VectorSubcoreMesh(core_axis_name="c", subcore_axis_name="s")

    @functools.partial(
        pl.kernel,
        mesh=mesh,
        out_type=jax.ShapeDtypeStruct((rows, d), dtype),
        scratch_types=[
            pltpu.VMEM_SHARED((2, _PHASE, d), dtype),
        ],
    )
    def sc_copy(in_hbm, out_hbm, spmem):
        c = lax.axis_index("c")
        s = lax.axis_index("s")
        core_base = c * rows_per_core
        is_reader = s < _RD_TILES

        for p in range(nph + 1):
            if p < nph:
                @pl.when(is_reader)
                def _():
                    off = core_base + p * _PHASE + s * rows_rd
                    pltpu.sync_copy(
                        in_hbm.at[pl.ds(off, rows_rd)],
                        spmem.at[p % 2, pl.ds(s * rows_rd, rows_rd)])
            if p >= 1:
                @pl.when(jnp.logical_not(is_reader))
                def _():
                    w = s - _RD_TILES
                    off = core_base + (p - 1) * _PHASE + w * rows_wr
                    pltpu.sync_copy(
                        spmem.at[(p - 1) % 2, pl.ds(w * rows_wr, rows_wr)],
                        out_hbm.at[pl.ds(off, rows_wr)])
            plsc.subcore_barrier()

    return sc_copy


def kernel(sequences, batch_first, padding_value, padding_side):
    B, L, D = sequences.shape
    rows = B * L
    flat = sequences.reshape(rows, D)
    out = _make_sc_copy(rows, D, sequences.dtype)(flat)
    return out.reshape(B, L, D)


# speedup vs baseline: 1.1521x; 1.1521x over previous
import functools

import jax
import jax.numpy as jnp
from jax import lax
from jax.experimental import pallas as pl
from jax.experimental.pallas import tpu as pltpu
from jax.experimental.pallas import tpu_sc as plsc

_NC = 2
_NS = 16
_NW = _NC * _NS
_CHUNK = 56
_NBUF = 2


def _make_sc_copy(rows, d, dtype):
    rows_per_w = rows // _NW
    sizes = []
    off = 0
    while off < rows_per_w:
        sz = min(_CHUNK, rows_per_w - off)
        sizes.append((off, sz))
        off += sz
    nch = len(sizes)
    nbuf = _NBUF
    mesh = plsc.VectorSubcoreMesh(core_axis_name="c", subcore_axis_name="s")

    @functools.partial(
        pl.kernel,
        mesh=mesh,
        out_type=jax.ShapeDtypeStruct((rows, d), dtype),
        scratch_types=(
            [pltpu.VMEM((_CHUNK, d), dtype) for _ in range(nbuf)]
            + [pltpu.SemaphoreType.DMA for _ in range(2 * nbuf)]
        ),
    )
    def sc_copy(in_hbm, out_hbm, *scratch):
        bufs = scratch[:nbuf]
        rsems = scratch[nbuf:2 * nbuf]
        wsems = scratch[2 * nbuf:]
        wid = lax.axis_index("s") * _NC + lax.axis_index("c")
        base = wid * rows_per_w

        def rd(i):
            off, sz = sizes[i]
            return pltpu.make_async_copy(
                in_hbm.at[pl.ds(base + off, sz)],
                bufs[i % nbuf].at[pl.ds(0, sz)], rsems[i % nbuf])

        def wr(i):
            off, sz = sizes[i]
            return pltpu.make_async_copy(
                bufs[i % nbuf].at[pl.ds(0, sz)],
                out_hbm.at[pl.ds(base + off, sz)], wsems[i % nbuf])

        for k in range(min(nbuf - 1, nch)):
            rd(k).start()
        for i in range(nch):
            j = i + nbuf - 1
            if j < nch:
                if j - nbuf >= 0:
                    wr(j - nbuf).wait()
                rd(j).start()
            rd(i).wait()
            wr(i).start()
        for k in range(max(0, nch - nbuf), nch):
            wr(k).wait()

    return sc_copy


def kernel(sequences, batch_first, padding_value, padding_side):
    B, L, D = sequences.shape
    rows = B * L
    flat = sequences.reshape(rows, D)
    out = _make_sc_copy(rows, D, sequences.dtype)(flat)
    return out.reshape(B, L, D)
